# Initial kernel scaffold; baseline (speedup 1.0000x reference)
import jax
import jax.numpy as jnp
from jax.experimental import pallas as pl


def kernel(c0, c1, c2, c3, c4, c5, c6, c7, c8, c9, c10, c11, c12, d0, d1, d2, d3, d4, d5, d6, d7, d8, d9, d10, d11, d12, d13, d14, d15, d16, d17, d18, d19, d20, d21, d22, d23, d24, d25, W1, b1, W2, b2, Wcomb, bcomb, E0, E1, E2, E3, E4, E5, E6, E7, E8, E9, E10, E11, E12, E13, E14, E15, E16, E17, E18, E19, E20, E21, E22, E23, E24, E25):
    raise NotImplementedError("write your pallas kernel here")


# speedup vs baseline: 1.2318x; 1.2318x over previous
import functools

import jax
import jax.numpy as jnp
from jax import lax
from jax.experimental import pallas as pl
from jax.experimental.pallas import tpu as pltpu
from jax.experimental.pallas import tpu_sc as plsc

NUM_CF = 13
NUM_DF = 26
EMBED_DIM = 128
VOCAB = 17
B = 16384

NC = 2
NS = 16
L = 16
NW = NC * NS
ROWS_W = B // NW
CHUNK = 256
NCHUNK = ROWS_W // CHUNK
NGROUP = CHUNK // L
TROWS = NUM_DF * VOCAB
FIELDS_PER_PASS = 13


def _tc_mlp_body(cf_ref, w1_ref, b1_ref, w2_ref, b2_ref, wb_ref, out_ref):
    cf = cf_ref[...]
    cf = jnp.where(jnp.isnan(cf), jnp.float32(0.0), cf)
    h = jnp.dot(cf, w1_ref[...], preferred_element_type=jnp.float32,
                precision=jax.lax.Precision.HIGHEST)
    h = jnp.maximum(h + b1_ref[...], 0.0)
    o = jnp.dot(h, w2_ref[...], preferred_element_type=jnp.float32,
                precision=jax.lax.Precision.HIGHEST)
    o = jnp.maximum(o + b2_ref[...], 0.0)
    out_ref[...] = o * wb_ref[0, 0] + wb_ref[0, 1]


def _tc_mlp(cf, w1, b1, w2, b2, wb):
    return pl.pallas_call(
        _tc_mlp_body,
        out_shape=jax.ShapeDtypeStruct((B, EMBED_DIM), jnp.float32),
        in_specs=[
            pl.BlockSpec(memory_space=pltpu.VMEM),
            pl.BlockSpec(memory_space=pltpu.VMEM),
            pl.BlockSpec(memory_space=pltpu.VMEM),
            pl.BlockSpec(memory_space=pltpu.VMEM),
            pl.BlockSpec(memory_space=pltpu.VMEM),
            pl.BlockSpec(memory_space=pltpu.SMEM),
        ],
        out_specs=pl.BlockSpec(memory_space=pltpu.VMEM),
    )(cf, w1, b1, w2, b2, wb)


def _sc_emb_body(t_hbm, d_hbm, w_hbm, mlp_hbm, out_hbm, t_v, d_v, w_v, buf):
    wid = lax.axis_index("s") * NC + lax.axis_index("c")
    base = wid * ROWS_W

    pltpu.sync_copy(t_hbm, t_v)
    pltpu.sync_copy(w_hbm, w_v)
    pltpu.sync_copy(d_hbm.at[:, pl.ds(base, ROWS_W)], d_v)

    iota = lax.iota(jnp.int32, L)

    for chunk in range(NCHUNK):
        cbase = (base + chunk * CHUNK) * EMBED_DIM
        pltpu.sync_copy(mlp_hbm.at[pl.ds(cbase, CHUNK * EMBED_DIM)], buf)

        for p in range(NUM_DF // FIELDS_PER_PASS):
            fields = range(p * FIELDS_PER_PASS, (p + 1) * FIELDS_PER_PASS)

            def g_body(g, _, fields=fields, chunk=chunk):
                sl = chunk * CHUNK + g * L
                dst = (g * L + iota) * EMBED_DIM
                rows = []
                wvs = []
                for i in fields:
                    dv = d_v[i, pl.ds(sl, L)]
                    rows.append((dv + 17 * i) * EMBED_DIM)
                    wvs.append(plsc.load_gather(
                        w_v, [jnp.full((L,), i + 1, jnp.int32)]))

                def c_body(c, _):
                    acc = jnp.zeros((L,), jnp.float32)
                    for ri, wv in zip(rows, wvs):
                        acc = acc + wv * plsc.load_gather(t_v, [ri + c])
                    plsc.addupdate_scatter(buf, [dst + c], acc)
                    return 0

                lax.fori_loop(0, EMBED_DIM, c_body, 0)
                return 0

            lax.fori_loop(0, NGROUP, g_body, 0)

        pltpu.sync_copy(buf, out_hbm.at[pl.ds(cbase, CHUNK * EMBED_DIM)])


_sc_emb = pl.kernel(
    _sc_emb_body,
    out_type=jax.ShapeDtypeStruct((B * EMBED_DIM,), jnp.float32),
    mesh=plsc.VectorSubcoreMesh(
        core_axis_name="c", subcore_axis_name="s",
        num_cores=NC, num_subcores=NS),
    scratch_types=[
        pltpu.VMEM((TROWS * EMBED_DIM,), jnp.float32),
        pltpu.VMEM((NUM_DF, ROWS_W), jnp.int32),
        pltpu.VMEM((2 * L,), jnp.float32),
        pltpu.VMEM((CHUNK * EMBED_DIM,), jnp.float32),
    ],
    compiler_params=pltpu.CompilerParams(needs_layout_passes=False),
)


def kernel(c0, c1, c2, c3, c4, c5, c6, c7, c8, c9, c10, c11, c12,
           d0, d1, d2, d3, d4, d5, d6, d7, d8, d9, d10, d11, d12,
           d13, d14, d15, d16, d17, d18, d19, d20, d21, d22, d23, d24, d25,
           W1, b1, W2, b2, Wcomb, bcomb,
           E0, E1, E2, E3, E4, E5, E6, E7, E8, E9, E10, E11, E12,
           E13, E14, E15, E16, E17, E18, E19, E20, E21, E22, E23, E24, E25):
    kw = dict(locals())
    cf = jnp.stack([kw["c%d" % i] for i in range(NUM_CF)], axis=1)
    d_all = jnp.stack([kw["d%d" % i] for i in range(NUM_DF)], axis=0)
    t_all = jnp.concatenate(
        [kw["E%d" % i] for i in range(NUM_DF)], axis=0).reshape(-1)
    w_pad = jnp.zeros((2 * L,), jnp.float32).at[:NUM_DF + 1].set(Wcomb[:, 0])
    wb = jnp.stack([Wcomb[0, 0], bcomb[0]]).reshape(1, 2)

    mlp_part = _tc_mlp(cf, W1, b1.reshape(1, -1), W2, b2.reshape(1, -1), wb)
    out = _sc_emb(t_all, d_all, w_pad, mlp_part.reshape(-1))
    return out.reshape(B, EMBED_DIM)


# speedup vs baseline: 1.4820x; 1.2031x over previous
import functools

import jax
import jax.numpy as jnp
from jax import lax
from jax.experimental import pallas as pl
from jax.experimental.pallas import tpu as pltpu
from jax.experimental.pallas import tpu_sc as plsc

NUM_CF = 13
NUM_DF = 26
EMBED_DIM = 128
VOCAB = 17
B = 16384

NC = 2
NS = 16
L = 16
NW = NC * NS
ROWS_W = B // NW
CHUNK = 256
NCHUNK = ROWS_W // CHUNK
NGROUP = CHUNK // L
TROWS = NUM_DF * VOCAB
FIELDS_PER_PASS = 13


def _tc_mlp_body(cf_ref, w1_ref, b1_ref, w2_ref, b2_ref, wb_ref, out_ref):
    cf = cf_ref[...]
    cf = jnp.where(jnp.isnan(cf), jnp.float32(0.0), cf)
    h = jnp.dot(cf, w1_ref[...], preferred_element_type=jnp.float32,
                precision=jax.lax.Precision.HIGHEST)
    h = jnp.maximum(h + b1_ref[...], 0.0)
    o = jnp.dot(h, w2_ref[...], preferred_element_type=jnp.float32,
                precision=jax.lax.Precision.HIGHEST)
    o = jnp.maximum(o + b2_ref[...], 0.0)
    out_ref[...] = o * wb_ref[0, 0] + wb_ref[0, 1]


def _tc_mlp(cf, w1, b1, w2, b2, wb):
    return pl.pallas_call(
        _tc_mlp_body,
        out_shape=jax.ShapeDtypeStruct((B, EMBED_DIM), jnp.float32),
        in_specs=[
            pl.BlockSpec(memory_space=pltpu.VMEM),
            pl.BlockSpec(memory_space=pltpu.VMEM),
            pl.BlockSpec(memory_space=pltpu.VMEM),
            pl.BlockSpec(memory_space=pltpu.VMEM),
            pl.BlockSpec(memory_space=pltpu.VMEM),
            pl.BlockSpec(memory_space=pltpu.SMEM),
        ],
        out_specs=pl.BlockSpec(memory_space=pltpu.VMEM),
    )(cf, w1, b1, w2, b2, wb)


def _sc_emb_body(t_hbm, d_hbm, w_hbm, mlp_hbm, out_hbm, t_v, d_v, w_v, buf):
    wid = lax.axis_index("s") * NC + lax.axis_index("c")
    base = wid * ROWS_W

    pltpu.sync_copy(t_hbm, t_v)
    pltpu.sync_copy(w_hbm, w_v)
    pltpu.sync_copy(d_hbm.at[:, pl.ds(base, ROWS_W)], d_v)

    iota = lax.iota(jnp.int32, L)

    for chunk in range(NCHUNK):
        cbase = (base + chunk * CHUNK) * EMBED_DIM
        pltpu.sync_copy(mlp_hbm.at[pl.ds(cbase, CHUNK * EMBED_DIM)], buf)

        for p in range(NUM_DF // FIELDS_PER_PASS):
            fields = range(p * FIELDS_PER_PASS, (p + 1) * FIELDS_PER_PASS)

            def g_body(g, _, fields=fields, chunk=chunk):
                sl = chunk * CHUNK + g * L
                dst = (g * L + iota) * EMBED_DIM
                rows = []
                wvs = []
                for i in fields:
                    dv = d_v[i, pl.ds(sl, L)]
                    rows.append((dv + 17 * i) * EMBED_DIM)
                    wvs.append(plsc.load_gather(
                        w_v, [jnp.full((L,), i + 1, jnp.int32)]))

                @plsc.parallel_loop(0, EMBED_DIM, unroll=4)
                def c_body(c):
                    acc = jnp.zeros((L,), jnp.float32)
                    for ri, wv in zip(rows, wvs):
                        acc = acc + wv * plsc.load_gather(t_v, [ri + c])
                    plsc.addupdate_scatter(buf, [dst + c], acc)

                return 0

            lax.fori_loop(0, NGROUP, g_body, 0)

        pltpu.sync_copy(buf, out_hbm.at[pl.ds(cbase, CHUNK * EMBED_DIM)])


_sc_emb = pl.kernel(
    _sc_emb_body,
    out_type=jax.ShapeDtypeStruct((B * EMBED_DIM,), jnp.float32),
    mesh=plsc.VectorSubcoreMesh(
        core_axis_name="c", subcore_axis_name="s",
        num_cores=NC, num_subcores=NS),
    scratch_types=[
        pltpu.VMEM((TROWS * EMBED_DIM,), jnp.float32),
        pltpu.VMEM((NUM_DF, ROWS_W), jnp.int32),
        pltpu.VMEM((2 * L,), jnp.float32),
        pltpu.VMEM((CHUNK * EMBED_DIM,), jnp.float32),
    ],
    compiler_params=pltpu.CompilerParams(needs_layout_passes=False),
)


def kernel(c0, c1, c2, c3, c4, c5, c6, c7, c8, c9, c10, c11, c12,
           d0, d1, d2, d3, d4, d5, d6, d7, d8, d9, d10, d11, d12,
           d13, d14, d15, d16, d17, d18, d19, d20, d21, d22, d23, d24, d25,
           W1, b1, W2, b2, Wcomb, bcomb,
           E0, E1, E2, E3, E4, E5, E6, E7, E8, E9, E10, E11, E12,
           E13, E14, E15, E16, E17, E18, E19, E20, E21, E22, E23, E24, E25):
    kw = dict(locals())
    cf = jnp.stack([kw["c%d" % i] for i in range(NUM_CF)], axis=1)
    d_all = jnp.stack([kw["d%d" % i] for i in range(NUM_DF)], axis=0)
    t_all = jnp.concatenate(
        [kw["E%d" % i] for i in range(NUM_DF)], axis=0).reshape(-1)
    w_pad = jnp.zeros((2 * L,), jnp.float32).at[:NUM_DF + 1].set(Wcomb[:, 0])
    wb = jnp.stack([Wcomb[0, 0], bcomb[0]]).reshape(1, 2)

    mlp_part = _tc_mlp(cf, W1, b1.reshape(1, -1), W2, b2.reshape(1, -1), wb)
    out = _sc_emb(t_all, d_all, w_pad, mlp_part.reshape(-1))
    return out.reshape(B, EMBED_DIM)


# speedup vs baseline: 2.5318x; 1.7083x over previous
import functools

import jax
import jax.numpy as jnp
from jax import lax
from jax.experimental import pallas as pl
from jax.experimental.pallas import tpu as pltpu
from jax.experimental.pallas import tpu_sc as plsc

NUM_CF = 13
NUM_DF = 26
EMBED_DIM = 128
VOCAB = 17
B = 16384

NC = 2
NS = 16
L = 16
NW = NC * NS
ROWS_W = B // NW
CHUNK = 256
NCHUNK = ROWS_W // CHUNK
NGROUP = CHUNK // L
TROWS = NUM_DF * VOCAB
HALF = EMBED_DIM // 2

MLP_BLK = 2048


def _tc_mlp_body(cf_ref, w1_ref, b1_ref, w2_ref, b2_ref, wb_ref, out_ref):
    cf = cf_ref[...]
    cf = jnp.where(jnp.isnan(cf), jnp.float32(0.0), cf)
    h = jnp.dot(cf, w1_ref[...], preferred_element_type=jnp.float32)
    h = jnp.maximum(h + b1_ref[...], 0.0)
    o = jnp.dot(h, w2_ref[...], preferred_element_type=jnp.float32)
    o = jnp.maximum(o + b2_ref[...], 0.0)
    out_ref[...] = o * wb_ref[0, 0] + wb_ref[0, 1]


def _tc_mlp(cf, w1, b1, w2, b2, wb):
    return pl.pallas_call(
        _tc_mlp_body,
        grid=(B // MLP_BLK,),
        out_shape=jax.ShapeDtypeStruct((B, EMBED_DIM), jnp.float32),
        in_specs=[
            pl.BlockSpec((MLP_BLK, NUM_CF), lambda i: (i, 0)),
            pl.BlockSpec((NUM_CF, 2 * NUM_CF), lambda i: (0, 0)),
            pl.BlockSpec((1, 2 * NUM_CF), lambda i: (0, 0)),
            pl.BlockSpec((2 * NUM_CF, EMBED_DIM), lambda i: (0, 0)),
            pl.BlockSpec((1, EMBED_DIM), lambda i: (0, 0)),
            pl.BlockSpec(memory_space=pltpu.SMEM),
        ],
        out_specs=pl.BlockSpec((MLP_BLK, EMBED_DIM), lambda i: (i, 0)),
    )(cf, w1, b1, w2, b2, wb)


def _sc_emb_body(tp_hbm, d_hbm, mlp_hbm, out_hbm,
                 tp_v, d_v, buf0, buf1, s0, s1, s2, s3):
    wid = lax.axis_index("s") * NC + lax.axis_index("c")
    base = wid * ROWS_W

    bufs = (buf0, buf1)
    isems = (s0, s1)
    osems = (s2, s3)

    incps = []
    for chunk in range(NCHUNK):
        cbase = (base + chunk * CHUNK) * EMBED_DIM
        incps.append(pltpu.async_copy(
            mlp_hbm.at[pl.ds(cbase, CHUNK * EMBED_DIM)],
            bufs[chunk], isems[chunk]))

    pltpu.sync_copy(tp_hbm, tp_v)
    pltpu.sync_copy(d_hbm.at[:, pl.ds(base, ROWS_W)], d_v)

    iota = lax.iota(jnp.int32, L)
    outcps = []
    for chunk in range(NCHUNK):
        cbase = (base + chunk * CHUNK) * EMBED_DIM
        incps[chunk].wait()

        def g_body(g, _, chunk=chunk, buf=bufs[chunk]):
            sl = chunk * CHUNK + g * L
            dst = (g * L + iota) * EMBED_DIM
            rows = []
            for i in range(NUM_DF):
                dv = d_v[i, pl.ds(sl, L)]
                rows.append((dv + VOCAB * i) * HALF)

            @plsc.parallel_loop(0, HALF, unroll=2)
            def c_body(c):
                acc_a = jnp.zeros((L,), jnp.float32)
                acc_b = jnp.zeros((L,), jnp.float32)
                for ri in rows:
                    w = plsc.bitcast(
                        plsc.load_gather(tp_v, [ri + c]), jnp.bfloat16)
                    a, b = plsc.unpack(
                        w, format=plsc.PackFormat.INTERLEAVED,
                        preferred_element_type=jnp.float32)
                    acc_a = acc_a + a
                    acc_b = acc_b + b
                plsc.addupdate_scatter(buf, [dst + c], acc_a)
                plsc.addupdate_scatter(buf, [dst + c + HALF], acc_b)

            return 0

        lax.fori_loop(0, NGROUP, g_body, 0)
        outcps.append(pltpu.async_copy(
            bufs[chunk], out_hbm.at[pl.ds(cbase, CHUNK * EMBED_DIM)],
            osems[chunk]))

    for cp in outcps:
        cp.wait()


_sc_emb = pl.kernel(
    _sc_emb_body,
    out_type=jax.ShapeDtypeStruct((B * EMBED_DIM,), jnp.float32),
    mesh=plsc.VectorSubcoreMesh(
        core_axis_name="c", subcore_axis_name="s",
        num_cores=NC, num_subcores=NS),
    scratch_types=[
        pltpu.VMEM((TROWS * HALF,), jnp.int32),
        pltpu.VMEM((NUM_DF, ROWS_W), jnp.int32),
        pltpu.VMEM((CHUNK * EMBED_DIM,), jnp.float32),
        pltpu.VMEM((CHUNK * EMBED_DIM,), jnp.float32),
        pltpu.SemaphoreType.DMA,
        pltpu.SemaphoreType.DMA,
        pltpu.SemaphoreType.DMA,
        pltpu.SemaphoreType.DMA,
    ],
    compiler_params=pltpu.CompilerParams(needs_layout_passes=False),
)


def kernel(c0, c1, c2, c3, c4, c5, c6, c7, c8, c9, c10, c11, c12,
           d0, d1, d2, d3, d4, d5, d6, d7, d8, d9, d10, d11, d12,
           d13, d14, d15, d16, d17, d18, d19, d20, d21, d22, d23, d24, d25,
           W1, b1, W2, b2, Wcomb, bcomb,
           E0, E1, E2, E3, E4, E5, E6, E7, E8, E9, E10, E11, E12,
           E13, E14, E15, E16, E17, E18, E19, E20, E21, E22, E23, E24, E25):
    kw = dict(locals())
    cf = jnp.stack([kw["c%d" % i] for i in range(NUM_CF)], axis=1)
    d_all = jnp.stack([kw["d%d" % i] for i in range(NUM_DF)], axis=0)

    t_all = jnp.concatenate(
        [kw["E%d" % i] for i in range(NUM_DF)], axis=0)
    wrep = jnp.repeat(Wcomb[1:, 0], VOCAB)[:, None]
    tb = (t_all * wrep).astype(jnp.bfloat16)
    pair = jnp.stack([tb[:, :HALF], tb[:, HALF:]], axis=-1)
    tp = lax.bitcast_convert_type(pair, jnp.int32).reshape(-1)

    wb = jnp.stack([Wcomb[0, 0], bcomb[0]]).reshape(1, 2)
    mlp_part = _tc_mlp(cf, W1, b1.reshape(1, -1), W2, b2.reshape(1, -1), wb)
    out = _sc_emb(tp, d_all, mlp_part.reshape(-1))
    return out.reshape(B, EMBED_DIM)


# speedup vs baseline: 8.3384x; 3.2935x over previous
import functools

import jax
import jax.numpy as jnp
from jax import lax
from jax.experimental import pallas as pl
from jax.experimental.pallas import tpu as pltpu
from jax.experimental.pallas import tpu_sc as plsc

NUM_CF = 13
NUM_DF = 26
EMBED_DIM = 128
VOCAB = 17
B = 16384

NC = 2
NS = 16
L = 16
NW = NC * NS
ROWS_W = B // NW
CHUNK = 256
NCHUNK = ROWS_W // CHUNK
NGROUP = CHUNK // L
TROWS = NUM_DF * VOCAB
HALF = EMBED_DIM // 2
TSTR = HALF + 1

MLP_BLK = 2048


def _tc_mlp_body(cf_ref, w1_ref, b1_ref, w2_ref, b2_ref, wb_ref, out_ref):
    cf = cf_ref[...]
    cf = jnp.where(jnp.isnan(cf), jnp.float32(0.0), cf)
    h = jnp.dot(cf, w1_ref[...], preferred_element_type=jnp.float32)
    h = jnp.maximum(h + b1_ref[...], 0.0)
    o = jnp.dot(h, w2_ref[...], preferred_element_type=jnp.float32)
    o = jnp.maximum(o + b2_ref[...], 0.0)
    out_ref[...] = o * wb_ref[0, 0] + wb_ref[0, 1]


def _tc_mlp(cf, w1, b1, w2, b2, wb):
    return pl.pallas_call(
        _tc_mlp_body,
        grid=(B // MLP_BLK,),
        out_shape=jax.ShapeDtypeStruct((B, EMBED_DIM), jnp.float32),
        in_specs=[
            pl.BlockSpec((MLP_BLK, NUM_CF), lambda i: (i, 0)),
            pl.BlockSpec((NUM_CF, 2 * NUM_CF), lambda i: (0, 0)),
            pl.BlockSpec((1, 2 * NUM_CF), lambda i: (0, 0)),
            pl.BlockSpec((2 * NUM_CF, EMBED_DIM), lambda i: (0, 0)),
            pl.BlockSpec((1, EMBED_DIM), lambda i: (0, 0)),
            pl.BlockSpec(memory_space=pltpu.SMEM),
        ],
        out_specs=pl.BlockSpec((MLP_BLK, EMBED_DIM), lambda i: (i, 0)),
    )(cf, w1, b1, w2, b2, wb)


def _sc_emb_body(tp_hbm, d_hbm, mlp_hbm, out_hbm,
                 tp_v, d_v, buf0, buf1, s0, s1, s2, s3):
    wid = lax.axis_index("s") * NC + lax.axis_index("c")
    base = wid * ROWS_W

    bufs = (buf0, buf1)
    isems = (s0, s1)
    osems = (s2, s3)

    incps = []
    for chunk in range(NCHUNK):
        cbase = (base + chunk * CHUNK) * EMBED_DIM
        incps.append(pltpu.async_copy(
            mlp_hbm.at[pl.ds(cbase, CHUNK * EMBED_DIM)],
            bufs[chunk], isems[chunk]))

    pltpu.sync_copy(tp_hbm, tp_v)
    pltpu.sync_copy(d_hbm.at[:, pl.ds(base, ROWS_W)], d_v)

    iota = lax.iota(jnp.int32, L)
    outcps = []
    for chunk in range(NCHUNK):
        cbase = (base + chunk * CHUNK) * EMBED_DIM
        incps[chunk].wait()

        def g_body(g, _, chunk=chunk, buf=bufs[chunk]):
            sl = chunk * CHUNK + g * L
            dst = (g * L + iota) * EMBED_DIM
            rows = []
            for i in range(NUM_DF):
                dv = d_v[i, pl.ds(sl, L)]
                rows.append((dv + VOCAB * i) * TSTR)

            @plsc.parallel_loop(0, HALF, unroll=2)
            def c_body(c):
                acc_a = jnp.zeros((L,), jnp.float32)
                acc_b = jnp.zeros((L,), jnp.float32)
                for ri in rows:
                    w = plsc.bitcast(
                        plsc.load_gather(tp_v, [ri + c]), jnp.bfloat16)
                    a, b = plsc.unpack(
                        w, format=plsc.PackFormat.INTERLEAVED,
                        preferred_element_type=jnp.float32)
                    acc_a = acc_a + a
                    acc_b = acc_b + b
                plsc.addupdate_scatter(buf, [dst + c], acc_a)
                plsc.addupdate_scatter(buf, [dst + c + HALF], acc_b)

            return 0

        lax.fori_loop(0, NGROUP, g_body, 0)
        outcps.append(pltpu.async_copy(
            bufs[chunk], out_hbm.at[pl.ds(cbase, CHUNK * EMBED_DIM)],
            osems[chunk]))

    for cp in outcps:
        cp.wait()


_sc_emb = pl.kernel(
    _sc_emb_body,
    out_type=jax.ShapeDtypeStruct((B * EMBED_DIM,), jnp.float32),
    mesh=plsc.VectorSubcoreMesh(
        core_axis_name="c", subcore_axis_name="s",
        num_cores=NC, num_subcores=NS),
    scratch_types=[
        pltpu.VMEM((TROWS * TSTR,), jnp.int32),
        pltpu.VMEM((NUM_DF, ROWS_W), jnp.int32),
        pltpu.VMEM((CHUNK * EMBED_DIM,), jnp.float32),
        pltpu.VMEM((CHUNK * EMBED_DIM,), jnp.float32),
        pltpu.SemaphoreType.DMA,
        pltpu.SemaphoreType.DMA,
        pltpu.SemaphoreType.DMA,
        pltpu.SemaphoreType.DMA,
    ],
    compiler_params=pltpu.CompilerParams(needs_layout_passes=False),
)


def kernel(c0, c1, c2, c3, c4, c5, c6, c7, c8, c9, c10, c11, c12,
           d0, d1, d2, d3, d4, d5, d6, d7, d8, d9, d10, d11, d12,
           d13, d14, d15, d16, d17, d18, d19, d20, d21, d22, d23, d24, d25,
           W1, b1, W2, b2, Wcomb, bcomb,
           E0, E1, E2, E3, E4, E5, E6, E7, E8, E9, E10, E11, E12,
           E13, E14, E15, E16, E17, E18, E19, E20, E21, E22, E23, E24, E25):
    kw = dict(locals())
    cf = jnp.stack([kw["c%d" % i] for i in range(NUM_CF)], axis=1)
    d_all = jnp.stack([kw["d%d" % i] for i in range(NUM_DF)], axis=0)

    t_all = jnp.concatenate(
        [kw["E%d" % i] for i in range(NUM_DF)], axis=0)
    wrep = jnp.repeat(Wcomb[1:, 0], VOCAB)[:, None]
    tb = (t_all * wrep).astype(jnp.bfloat16)
    pair = jnp.stack([tb[:, :HALF], tb[:, HALF:]], axis=-1)
    tp = lax.bitcast_convert_type(pair, jnp.int32)
    tp = jnp.pad(tp, ((0, 0), (0, TSTR - HALF))).reshape(-1)

    wb = jnp.stack([Wcomb[0, 0], bcomb[0]]).reshape(1, 2)
    mlp_part = _tc_mlp(cf, W1, b1.reshape(1, -1), W2, b2.reshape(1, -1), wb)
    out = _sc_emb(tp, d_all, mlp_part.reshape(-1))
    return out.reshape(B, EMBED_DIM)


# speedup vs baseline: 8.6349x; 1.0356x over previous
import functools

import jax
import jax.numpy as jnp
from jax import lax
from jax.experimental import pallas as pl
from jax.experimental.pallas import tpu as pltpu
from jax.experimental.pallas import tpu_sc as plsc

NUM_CF = 13
NUM_DF = 26
EMBED_DIM = 128
VOCAB = 17
B = 16384

NC = 2
NS = 16
L = 16
NW = NC * NS
ROWS_W = B // NW
CHUNK = 256
NCHUNK = ROWS_W // CHUNK
NGROUP = CHUNK // L
TROWS = NUM_DF * VOCAB
HALF = EMBED_DIM // 2
TSTR = HALF + 1

MLP_BLK = 2048


def _tc_mlp_body(cf_ref, w1_ref, b1_ref, w2_ref, b2_ref, wb_ref, out_ref):
    cf = cf_ref[...]
    cf = jnp.where(jnp.isnan(cf), jnp.float32(0.0), cf)
    h = jnp.dot(cf, w1_ref[...], preferred_element_type=jnp.float32)
    h = jnp.maximum(h + b1_ref[...], 0.0)
    o = jnp.dot(h, w2_ref[...], preferred_element_type=jnp.float32)
    o = jnp.maximum(o + b2_ref[...], 0.0)
    out_ref[...] = o * wb_ref[0, 0] + wb_ref[0, 1]


def _tc_mlp(cf, w1, b1, w2, b2, wb):
    return pl.pallas_call(
        _tc_mlp_body,
        grid=(B // MLP_BLK,),
        out_shape=jax.ShapeDtypeStruct((B, EMBED_DIM), jnp.float32),
        in_specs=[
            pl.BlockSpec((MLP_BLK, NUM_CF), lambda i: (i, 0)),
            pl.BlockSpec((NUM_CF, 2 * NUM_CF), lambda i: (0, 0)),
            pl.BlockSpec((1, 2 * NUM_CF), lambda i: (0, 0)),
            pl.BlockSpec((2 * NUM_CF, EMBED_DIM), lambda i: (0, 0)),
            pl.BlockSpec((1, EMBED_DIM), lambda i: (0, 0)),
            pl.BlockSpec(memory_space=pltpu.SMEM),
        ],
        out_specs=pl.BlockSpec((MLP_BLK, EMBED_DIM), lambda i: (i, 0)),
    )(cf, w1, b1, w2, b2, wb)


def _sc_emb_body(tp_hbm, d_hbm, mlp_hbm, out_hbm,
                 tp_v, d_v, buf0, buf1, s0, s1, s2, s3):
    wid = lax.axis_index("s") * NC + lax.axis_index("c")
    base = wid * ROWS_W

    bufs = (buf0, buf1)
    isems = (s0, s1)
    osems = (s2, s3)

    incps = []
    for chunk in range(NCHUNK):
        cbase = (base + chunk * CHUNK) * EMBED_DIM
        incps.append(pltpu.async_copy(
            mlp_hbm.at[pl.ds(cbase, CHUNK * EMBED_DIM)],
            bufs[chunk], isems[chunk]))

    pltpu.sync_copy(tp_hbm, tp_v)
    pltpu.sync_copy(d_hbm.at[:, pl.ds(base, ROWS_W)], d_v)

    iota = lax.iota(jnp.int32, L)
    outcps = []
    for chunk in range(NCHUNK):
        cbase = (base + chunk * CHUNK) * EMBED_DIM
        incps[chunk].wait()

        def g_body(g, _, chunk=chunk, buf=bufs[chunk]):
            sl = chunk * CHUNK + g * L
            dst = (g * L + iota) * EMBED_DIM
            rows = []
            for i in range(NUM_DF):
                dv = d_v[i, pl.ds(sl, L)]
                rows.append((dv + VOCAB * i) * TSTR)

            @plsc.parallel_loop(0, HALF, unroll=4)
            def c_body(c):
                acc_a = jnp.zeros((L,), jnp.float32)
                acc_b = jnp.zeros((L,), jnp.float32)
                for ri in rows:
                    w = plsc.bitcast(
                        plsc.load_gather(tp_v, [ri + c]), jnp.bfloat16)
                    a, b = plsc.unpack(
                        w, format=plsc.PackFormat.INTERLEAVED,
                        preferred_element_type=jnp.float32)
                    acc_a = acc_a + a
                    acc_b = acc_b + b
                plsc.addupdate_scatter(buf, [dst + c], acc_a)
                plsc.addupdate_scatter(buf, [dst + c + HALF], acc_b)

            return 0

        lax.fori_loop(0, NGROUP, g_body, 0)
        outcps.append(pltpu.async_copy(
            bufs[chunk], out_hbm.at[pl.ds(cbase, CHUNK * EMBED_DIM)],
            osems[chunk]))

    for cp in outcps:
        cp.wait()


_sc_emb = pl.kernel(
    _sc_emb_body,
    out_type=jax.ShapeDtypeStruct((B * EMBED_DIM,), jnp.float32),
    mesh=plsc.VectorSubcoreMesh(
        core_axis_name="c", subcore_axis_name="s",
        num_cores=NC, num_subcores=NS),
    scratch_types=[
        pltpu.VMEM((TROWS * TSTR,), jnp.int32),
        pltpu.VMEM((NUM_DF, ROWS_W), jnp.int32),
        pltpu.VMEM((CHUNK * EMBED_DIM,), jnp.float32),
        pltpu.VMEM((CHUNK * EMBED_DIM,), jnp.float32),
        pltpu.SemaphoreType.DMA,
        pltpu.SemaphoreType.DMA,
        pltpu.SemaphoreType.DMA,
        pltpu.SemaphoreType.DMA,
    ],
    compiler_params=pltpu.CompilerParams(needs_layout_passes=False),
)


def kernel(c0, c1, c2, c3, c4, c5, c6, c7, c8, c9, c10, c11, c12,
           d0, d1, d2, d3, d4, d5, d6, d7, d8, d9, d10, d11, d12,
           d13, d14, d15, d16, d17, d18, d19, d20, d21, d22, d23, d24, d25,
           W1, b1, W2, b2, Wcomb, bcomb,
           E0, E1, E2, E3, E4, E5, E6, E7, E8, E9, E10, E11, E12,
           E13, E14, E15, E16, E17, E18, E19, E20, E21, E22, E23, E24, E25):
    kw = dict(locals())
    cf = jnp.stack([kw["c%d" % i] for i in range(NUM_CF)], axis=1)
    d_all = jnp.stack([kw["d%d" % i] for i in range(NUM_DF)], axis=0)

    t_all = jnp.concatenate(
        [kw["E%d" % i] for i in range(NUM_DF)], axis=0)
    wrep = jnp.repeat(Wcomb[1:, 0], VOCAB)[:, None]
    tb = (t_all * wrep).astype(jnp.bfloat16)
    pair = jnp.stack([tb[:, :HALF], tb[:, HALF:]], axis=-1)
    tp = lax.bitcast_convert_type(pair, jnp.int32)
    tp = jnp.pad(tp, ((0, 0), (0, TSTR - HALF))).reshape(-1)

    wb = jnp.stack([Wcomb[0, 0], bcomb[0]]).reshape(1, 2)
    mlp_part = _tc_mlp(cf, W1, b1.reshape(1, -1), W2, b2.reshape(1, -1), wb)
    out = _sc_emb(tp, d_all, mlp_part.reshape(-1))
    return out.reshape(B, EMBED_DIM)
